# Initial kernel scaffold; baseline (speedup 1.0000x reference)
import jax
import jax.numpy as jnp
from jax.experimental import pallas as pl


def kernel(z, edge_index, edge_attr, W, b):
    raise NotImplementedError("write your pallas kernel here")


# speedup vs baseline: 6.4598x; 6.4598x over previous
import functools

import jax
import jax.numpy as jnp
from jax import lax
from jax.experimental import pallas as pl
from jax.experimental.pallas import tpu as pltpu
from jax.experimental.pallas import tpu_sc as plsc

N = 4096
E = 131072
LAT = 256
NC = 2
NS = 16
L = 16

EPT = E // NS
CH = 128
NJ = EPT // CH
ROWS_PER_BLK = 256
BLK_WORDS = ROWS_PER_BLK * N
DUMP = 512
BLKS_PER_SC = N // ROWS_PER_BLK // NC
SLICE = BLK_WORDS // NS
ZCH = 8192


def _tc_prep(z_ref, w_ref, b_ref, ea_ref, u_ref, v_ref, es_ref):
    zz = z_ref[...]
    w = w_ref[...]
    u_ref[...] = (
        jnp.dot(zz, w[:LAT, :], preferred_element_type=jnp.float32)
        + b_ref[...]
    )
    v_ref[...] = jnp.dot(zz, w[LAT:, :], preferred_element_type=jnp.float32)
    r_idx = lax.broadcasted_iota(jnp.int32, (16 * 128, 128), 0)
    c_idx = lax.broadcasted_iota(jnp.int32, (16 * 128, 128), 1)
    sel = (r_idx // 16 == c_idx).astype(jnp.float32)
    es_ref[...] = jnp.dot(
        ea_ref[...], sel, preferred_element_type=jnp.float32
    )


_mesh = plsc.VectorSubcoreMesh(core_axis_name="c", subcore_axis_name="s")


@functools.partial(
    pl.kernel,
    mesh=_mesh,
    out_type=jax.ShapeDtypeStruct((N * N,), jnp.float32),
    compiler_params=pltpu.CompilerParams(needs_layout_passes=False),
    scratch_types=[
        pltpu.VMEM_SHARED((BLK_WORDS + DUMP,), jnp.float32),
        pltpu.VMEM((N,), jnp.float32),
        pltpu.VMEM((N,), jnp.float32),
        pltpu.VMEM((EPT,), jnp.int32),
        pltpu.VMEM((EPT,), jnp.int32),
        pltpu.VMEM((EPT,), jnp.int32),
        pltpu.VMEM((EPT,), jnp.float32),
        pltpu.VMEM((EPT,), jnp.float32),
        pltpu.VMEM((ZCH,), jnp.float32),
    ],
)
def _sc_scatter(u_hbm, v_hbm, row_hbm, col_hbm, es_hbm, out_hbm,
                shared, u_ts, v_ts, idx_ts, col_ts, k_ts,
                p_ts, es_ts, zero_ts):
    c = lax.axis_index("c")
    s = lax.axis_index("s")

    pltpu.sync_copy(u_hbm, u_ts)
    pltpu.sync_copy(v_hbm, v_ts)
    pltpu.sync_copy(row_hbm.at[s], idx_ts)
    pltpu.sync_copy(col_hbm.at[s], col_ts)
    pltpu.sync_copy(es_hbm.at[s], es_ts)

    zeros16 = jnp.zeros((L,), jnp.float32)

    def zbody(i, _):
        zero_ts[pl.ds(i * L, L)] = zeros16
        return _

    lax.fori_loop(0, ZCH // L, zbody, None)

    def prep_body(i, _):
        o = i * L
        rv = idx_ts[pl.ds(o, L)]
        cv = col_ts[pl.ds(o, L)]
        k_ts[pl.ds(o, L)] = rv * N + cv
        uv = plsc.load_gather(u_ts, [rv])
        vv = plsc.load_gather(v_ts, [cv])
        p_ts[pl.ds(o, L)] = uv + vv
        return _

    lax.fori_loop(0, EPT // L, prep_body, None)

    for jj in range(BLKS_PER_SC):
        blk = c * BLKS_PER_SC + jj
        base = blk * BLK_WORDS

        def zc_body(i, _):
            pltpu.sync_copy(
                zero_ts, shared.at[pl.ds(s * SLICE + i * ZCH, ZCH)]
            )
            return _

        lax.fori_loop(0, SLICE // ZCH, zc_body, None)

        def idx_body(i, _):
            o = i * L
            kv = k_ts[pl.ds(o, L)]
            inblk = (kv >> 20) == blk
            local = kv & (BLK_WORDS - 1)
            dump = BLK_WORDS + (kv & (DUMP - 1))
            idx_ts[pl.ds(o, L)] = jnp.where(inblk, local, dump)
            return _

        lax.fori_loop(0, EPT // L, idx_body, None)
        plsc.subcore_barrier()

        pltpu.sync_copy(p_ts, shared.at[idx_ts])
        plsc.subcore_barrier()

        pltpu.sync_copy(es_ts, shared.at[idx_ts], add=True)
        plsc.subcore_barrier()

        pltpu.sync_copy(
            shared.at[pl.ds(s * SLICE, SLICE)],
            out_hbm.at[pl.ds(base + s * SLICE, SLICE)],
        )


def kernel(z, edge_index, edge_attr, W, b):
    row = edge_index[0].astype(jnp.int32)
    col = edge_index[1].astype(jnp.int32)
    u2, v2, es2 = pl.pallas_call(
        _tc_prep,
        out_shape=(
            jax.ShapeDtypeStruct((N, 1), jnp.float32),
            jax.ShapeDtypeStruct((N, 1), jnp.float32),
            jax.ShapeDtypeStruct((E // 128, 128), jnp.float32),
        ),
    )(z, W, b.reshape(1, 1), edge_attr.reshape(E // 128, 128 * 16))
    flat = _sc_scatter(
        u2.reshape(N),
        v2.reshape(N),
        row.reshape(NS, EPT),
        col.reshape(NS, EPT),
        es2.reshape(NS, EPT),
    )
    return flat.reshape(N, N)


# speedup vs baseline: 6.5020x; 1.0065x over previous
import functools

import jax
import jax.numpy as jnp
from jax import lax
from jax.experimental import pallas as pl
from jax.experimental.pallas import tpu as pltpu
from jax.experimental.pallas import tpu_sc as plsc

N = 4096
E = 131072
LAT = 256
NC = 2
NS = 16
L = 16

EPT = E // NS
CH = 128
NJ = EPT // CH
ROWS_PER_BLK = 256
BLK_WORDS = ROWS_PER_BLK * N
BLKS_PER_SC = N // ROWS_PER_BLK // NC
SLICE = BLK_WORDS // NS
ZCH = 8192


def _tc_prep(z_ref, w_ref, b_ref, ea_ref, u_ref, v_ref, es_ref):
    zz = z_ref[...]
    w = w_ref[...]
    u_ref[...] = (
        jnp.dot(zz, w[:LAT, :], preferred_element_type=jnp.float32)
        + b_ref[...]
    )
    v_ref[...] = jnp.dot(zz, w[LAT:, :], preferred_element_type=jnp.float32)
    r_idx = lax.broadcasted_iota(jnp.int32, (16 * 128, 128), 0)
    c_idx = lax.broadcasted_iota(jnp.int32, (16 * 128, 128), 1)
    sel = (r_idx // 16 == c_idx).astype(jnp.float32)
    es_ref[...] = jnp.dot(
        ea_ref[...], sel, preferred_element_type=jnp.float32
    )


_mesh = plsc.VectorSubcoreMesh(core_axis_name="c", subcore_axis_name="s")


@functools.partial(
    pl.kernel,
    mesh=_mesh,
    out_type=jax.ShapeDtypeStruct((N * N,), jnp.float32),
    compiler_params=pltpu.CompilerParams(needs_layout_passes=False),
    scratch_types=[
        pltpu.VMEM_SHARED((BLK_WORDS,), jnp.float32),
        pltpu.VMEM((N,), jnp.float32),
        pltpu.VMEM((N,), jnp.float32),
        pltpu.VMEM((EPT,), jnp.int32),
        pltpu.VMEM((EPT,), jnp.int32),
        pltpu.VMEM((EPT,), jnp.int32),
        pltpu.VMEM((EPT,), jnp.float32),
        pltpu.VMEM((EPT,), jnp.float32),
        pltpu.VMEM((ZCH,), jnp.float32),
    ],
)
def _sc_scatter(u_hbm, v_hbm, row_hbm, col_hbm, es_hbm, out_hbm,
                shared, u_ts, v_ts, idx_ts, col_ts, k_ts,
                p_ts, es_ts, zero_ts):
    c = lax.axis_index("c")
    s = lax.axis_index("s")

    pltpu.sync_copy(u_hbm, u_ts)
    pltpu.sync_copy(v_hbm, v_ts)
    pltpu.sync_copy(row_hbm.at[s], idx_ts)
    pltpu.sync_copy(col_hbm.at[s], col_ts)
    pltpu.sync_copy(es_hbm.at[s], es_ts)

    zeros16 = jnp.zeros((L,), jnp.float32)

    def zbody(i, _):
        zero_ts[pl.ds(i * L, L)] = zeros16
        return _

    lax.fori_loop(0, ZCH // L, zbody, None)

    def prep_body(i, _):
        o = i * L
        rv = idx_ts[pl.ds(o, L)]
        cv = col_ts[pl.ds(o, L)]
        k_ts[pl.ds(o, L)] = rv * N + cv
        uv = plsc.load_gather(u_ts, [rv])
        vv = plsc.load_gather(v_ts, [cv])
        p_ts[pl.ds(o, L)] = uv + vv
        return _

    lax.fori_loop(0, EPT // L, prep_body, None)

    for jj in range(BLKS_PER_SC):
        blk = c * BLKS_PER_SC + jj
        base = blk * BLK_WORDS

        def zc_body(i, _):
            pltpu.sync_copy(
                zero_ts, shared.at[pl.ds(s * SLICE + i * ZCH, ZCH)]
            )
            return _

        lax.fori_loop(0, SLICE // ZCH, zc_body, None)

        def idx_body(i, _):
            o = i * L
            kv = k_ts[pl.ds(o, L)]
            inblk = (kv >> 20) == blk
            local = kv & (BLK_WORDS - 1)
            idx_ts[pl.ds(o, L)] = jnp.where(inblk, local, -1)
            return _

        lax.fori_loop(0, EPT // L, idx_body, None)
        plsc.subcore_barrier()

        filtered = plsc.Indices(idx_ts, ignored_value=-1)
        pltpu.sync_copy(p_ts, shared.at[filtered])
        plsc.subcore_barrier()

        pltpu.sync_copy(es_ts, shared.at[filtered], add=True)
        plsc.subcore_barrier()

        pltpu.sync_copy(
            shared.at[pl.ds(s * SLICE, SLICE)],
            out_hbm.at[pl.ds(base + s * SLICE, SLICE)],
        )


def kernel(z, edge_index, edge_attr, W, b):
    row = edge_index[0].astype(jnp.int32)
    col = edge_index[1].astype(jnp.int32)
    u2, v2, es2 = pl.pallas_call(
        _tc_prep,
        out_shape=(
            jax.ShapeDtypeStruct((N, 1), jnp.float32),
            jax.ShapeDtypeStruct((N, 1), jnp.float32),
            jax.ShapeDtypeStruct((E // 128, 128), jnp.float32),
        ),
    )(z, W, b.reshape(1, 1), edge_attr.reshape(E // 128, 128 * 16))
    flat = _sc_scatter(
        u2.reshape(N),
        v2.reshape(N),
        row.reshape(NS, EPT),
        col.reshape(NS, EPT),
        es2.reshape(NS, EPT),
    )
    return flat.reshape(N, N)


# speedup vs baseline: 6.6284x; 1.0194x over previous
import functools

import jax
import jax.numpy as jnp
from jax import lax
from jax.experimental import pallas as pl
from jax.experimental.pallas import tpu as pltpu
from jax.experimental.pallas import tpu_sc as plsc

N = 4096
E = 131072
LAT = 256
D_EDGE = 16
NC = 2
NS = 16
L = 16

EPT = E // NS
CH = 128
NJ = EPT // CH
ROWS_PER_BLK = 256
BLK_WORDS = ROWS_PER_BLK * N
BLKS_PER_SC = N // ROWS_PER_BLK // NC
SLICE = BLK_WORDS // NS
ZCH = 8192


def _tc_prep(z_ref, w_ref, b_ref, ea_ref, u_ref, v_ref, es_ref):
    @pl.when(pl.program_id(0) == 0)
    def _():
        zz = z_ref[...]
        w = w_ref[...]
        u_ref[...] = (
            jnp.dot(zz, w[:LAT, :], preferred_element_type=jnp.float32)
            + b_ref[...]
        )
        v_ref[...] = jnp.dot(
            zz, w[LAT:, :], preferred_element_type=jnp.float32
        )

    ones = jnp.ones((1, D_EDGE), jnp.float32)
    es_ref[...] = lax.dot_general(
        ones,
        ea_ref[...],
        dimension_numbers=(((1,), (1,)), ((), ())),
        preferred_element_type=jnp.float32,
    ).reshape(1, 1, EPT)


_mesh = plsc.VectorSubcoreMesh(core_axis_name="c", subcore_axis_name="s")


@functools.partial(
    pl.kernel,
    mesh=_mesh,
    out_type=jax.ShapeDtypeStruct((N * N,), jnp.float32),
    compiler_params=pltpu.CompilerParams(needs_layout_passes=False),
    scratch_types=[
        pltpu.VMEM_SHARED((BLK_WORDS,), jnp.float32),
        pltpu.VMEM((N,), jnp.float32),
        pltpu.VMEM((N,), jnp.float32),
        pltpu.VMEM((EPT,), jnp.int32),
        pltpu.VMEM((EPT,), jnp.int32),
        pltpu.VMEM((EPT,), jnp.int32),
        pltpu.VMEM((EPT,), jnp.float32),
        pltpu.VMEM((EPT,), jnp.float32),
        pltpu.VMEM((ZCH,), jnp.float32),
    ],
)
def _sc_scatter(u_hbm, v_hbm, row_hbm, col_hbm, es_hbm, out_hbm,
                shared, u_ts, v_ts, idx_ts, col_ts, k_ts,
                p_ts, es_ts, zero_ts):
    c = lax.axis_index("c")
    s = lax.axis_index("s")

    pltpu.sync_copy(u_hbm, u_ts)
    pltpu.sync_copy(v_hbm, v_ts)
    pltpu.sync_copy(row_hbm.at[s], idx_ts)
    pltpu.sync_copy(col_hbm.at[s], col_ts)
    pltpu.sync_copy(es_hbm.at[s], es_ts)

    zeros16 = jnp.zeros((L,), jnp.float32)

    def zbody(i, _):
        zero_ts[pl.ds(i * L, L)] = zeros16
        return _

    lax.fori_loop(0, ZCH // L, zbody, None)

    def prep_body(i, _):
        o = i * L
        rv = idx_ts[pl.ds(o, L)]
        cv = col_ts[pl.ds(o, L)]
        k_ts[pl.ds(o, L)] = rv * N + cv
        uv = plsc.load_gather(u_ts, [rv])
        vv = plsc.load_gather(v_ts, [cv])
        p_ts[pl.ds(o, L)] = uv + vv
        return _

    lax.fori_loop(0, EPT // L, prep_body, None)

    for jj in range(BLKS_PER_SC):
        blk = c * BLKS_PER_SC + jj
        base = blk * BLK_WORDS

        def zc_body(i, _):
            pltpu.sync_copy(
                zero_ts, shared.at[pl.ds(s * SLICE + i * ZCH, ZCH)]
            )
            return _

        lax.fori_loop(0, SLICE // ZCH, zc_body, None)

        def idx_body(i, _):
            o = i * L
            kv = k_ts[pl.ds(o, L)]
            inblk = (kv >> 20) == blk
            local = kv & (BLK_WORDS - 1)
            idx_ts[pl.ds(o, L)] = jnp.where(inblk, local, -1)
            return _

        lax.fori_loop(0, EPT // L, idx_body, None)
        plsc.subcore_barrier()

        filtered = plsc.Indices(idx_ts, ignored_value=-1)
        pltpu.sync_copy(p_ts, shared.at[filtered])
        plsc.subcore_barrier()

        pltpu.sync_copy(es_ts, shared.at[filtered], add=True)
        plsc.subcore_barrier()

        pltpu.sync_copy(
            shared.at[pl.ds(s * SLICE, SLICE)],
            out_hbm.at[pl.ds(base + s * SLICE, SLICE)],
        )


def kernel(z, edge_index, edge_attr, W, b):
    row = edge_index[0].astype(jnp.int32)
    col = edge_index[1].astype(jnp.int32)
    u2, v2, es3 = pl.pallas_call(
        _tc_prep,
        grid=(NS,),
        in_specs=[
            pl.BlockSpec((N, LAT), lambda i: (0, 0)),
            pl.BlockSpec((2 * LAT, 1), lambda i: (0, 0)),
            pl.BlockSpec((1, 1), lambda i: (0, 0)),
            pl.BlockSpec((EPT, D_EDGE), lambda i: (i, 0)),
        ],
        out_specs=[
            pl.BlockSpec((N, 1), lambda i: (0, 0)),
            pl.BlockSpec((N, 1), lambda i: (0, 0)),
            pl.BlockSpec((1, 1, EPT), lambda i: (i, 0, 0)),
        ],
        out_shape=(
            jax.ShapeDtypeStruct((N, 1), jnp.float32),
            jax.ShapeDtypeStruct((N, 1), jnp.float32),
            jax.ShapeDtypeStruct((NS, 1, EPT), jnp.float32),
        ),
    )(z, W, b.reshape(1, 1), edge_attr)
    flat = _sc_scatter(
        u2.reshape(N),
        v2.reshape(N),
        row.reshape(NS, EPT),
        col.reshape(NS, EPT),
        es3.reshape(NS, EPT),
    )
    return flat.reshape(N, N)


# speedup vs baseline: 8.8008x; 1.3277x over previous
import functools

import jax
import jax.numpy as jnp
from jax import lax
from jax.experimental import pallas as pl
from jax.experimental.pallas import tpu as pltpu
from jax.experimental.pallas import tpu_sc as plsc

N = 4096
E = 131072
LAT = 256
D_EDGE = 16
NC = 2
NS = 16
L = 16

EPT = E // NS
CH = 128
NJ = EPT // CH
ROWS_PER_BLK = 256
BLK_WORDS = ROWS_PER_BLK * N
BLKS_PER_SC = N // ROWS_PER_BLK // NC
SLICE = BLK_WORDS // NS
ZCH = 8192


def _tc_prep(z_ref, w_ref, b_ref, ea_ref, u_ref, v_ref, es_ref):
    @pl.when(pl.program_id(0) == 0)
    def _():
        zz = z_ref[...]
        w = w_ref[...]
        u_ref[...] = (
            jnp.dot(zz, w[:LAT, :], preferred_element_type=jnp.float32)
            + b_ref[...]
        )
        v_ref[...] = jnp.dot(
            zz, w[LAT:, :], preferred_element_type=jnp.float32
        )

    ones = jnp.ones((1, D_EDGE), jnp.float32)
    es_ref[...] = lax.dot_general(
        ones,
        ea_ref[...],
        dimension_numbers=(((1,), (1,)), ((), ())),
        preferred_element_type=jnp.float32,
    ).reshape(1, 1, EPT)


_mesh = plsc.VectorSubcoreMesh(core_axis_name="c", subcore_axis_name="s")


@functools.partial(
    pl.kernel,
    mesh=_mesh,
    out_type=jax.ShapeDtypeStruct((N * N,), jnp.float32),
    compiler_params=pltpu.CompilerParams(needs_layout_passes=False),
    scratch_types=[
        pltpu.VMEM_SHARED((BLK_WORDS,), jnp.float32),
        pltpu.VMEM((N,), jnp.float32),
        pltpu.VMEM((N,), jnp.float32),
        pltpu.VMEM((EPT,), jnp.int32),
        pltpu.VMEM((EPT,), jnp.int32),
        pltpu.VMEM((EPT,), jnp.int32),
        pltpu.VMEM((EPT,), jnp.float32),
        pltpu.VMEM((EPT,), jnp.float32),
        pltpu.VMEM((ZCH,), jnp.float32),
    ],
)
def _sc_scatter(u_hbm, v_hbm, row_hbm, col_hbm, es_hbm, out_hbm,
                shared, u_ts, v_ts, idx_ts, col_ts, k_ts,
                p_ts, es_ts, zero_ts):
    c = lax.axis_index("c")
    s = lax.axis_index("s")

    pltpu.sync_copy(u_hbm, u_ts)
    pltpu.sync_copy(v_hbm, v_ts)
    pltpu.sync_copy(row_hbm.at[s], idx_ts)
    pltpu.sync_copy(col_hbm.at[s], col_ts)
    pltpu.sync_copy(es_hbm.at[s], es_ts)

    zeros16 = jnp.zeros((L,), jnp.float32)

    def zbody(i, _):
        zero_ts[pl.ds(i * L, L)] = zeros16
        return _

    lax.fori_loop(0, ZCH // L, zbody, None)

    def prep_body(i, _):
        o = i * L
        rv = idx_ts[pl.ds(o, L)]
        cv = col_ts[pl.ds(o, L)]
        k_ts[pl.ds(o, L)] = rv * N + cv
        uv = plsc.load_gather(u_ts, [rv])
        vv = plsc.load_gather(v_ts, [cv])
        p_ts[pl.ds(o, L)] = uv + vv
        return _

    lax.fori_loop(0, EPT // L, prep_body, None)

    for jj in range(BLKS_PER_SC):
        blk = c * BLKS_PER_SC + jj
        base = blk * BLK_WORDS

        def zc_body(i, _):
            pltpu.sync_copy(
                zero_ts, shared.at[pl.ds(s * SLICE + i * ZCH, ZCH)]
            )
            return _

        lax.fori_loop(0, SLICE // ZCH, zc_body, None)

        def idx_body(i, _):
            o = i * L
            kv = k_ts[pl.ds(o, L)]
            inblk = (kv >> 20) == blk
            rl = (kv >> 12) & (ROWS_PER_BLK - 1)
            cc = kv & (N - 1)
            local = (
                ((rl >> 3) << 15)
                + ((cc >> 7) << 10)
                + ((rl & 7) << 7)
                + (cc & 127)
            )
            idx_ts[pl.ds(o, L)] = jnp.where(inblk, local, -1)
            return _

        lax.fori_loop(0, EPT // L, idx_body, None)
        plsc.subcore_barrier()

        filtered = plsc.Indices(idx_ts, ignored_value=-1)
        pltpu.sync_copy(p_ts, shared.at[filtered])
        plsc.subcore_barrier()

        pltpu.sync_copy(es_ts, shared.at[filtered], add=True)
        plsc.subcore_barrier()

        pltpu.sync_copy(
            shared.at[pl.ds(s * SLICE, SLICE)],
            out_hbm.at[pl.ds(base + s * SLICE, SLICE)],
        )


def kernel(z, edge_index, edge_attr, W, b):
    row = edge_index[0].astype(jnp.int32)
    col = edge_index[1].astype(jnp.int32)
    u2, v2, es3 = pl.pallas_call(
        _tc_prep,
        grid=(NS,),
        in_specs=[
            pl.BlockSpec((N, LAT), lambda i: (0, 0)),
            pl.BlockSpec((2 * LAT, 1), lambda i: (0, 0)),
            pl.BlockSpec((1, 1), lambda i: (0, 0)),
            pl.BlockSpec((EPT, D_EDGE), lambda i: (i, 0)),
        ],
        out_specs=[
            pl.BlockSpec((N, 1), lambda i: (0, 0)),
            pl.BlockSpec((N, 1), lambda i: (0, 0)),
            pl.BlockSpec((1, 1, EPT), lambda i: (i, 0, 0)),
        ],
        out_shape=(
            jax.ShapeDtypeStruct((N, 1), jnp.float32),
            jax.ShapeDtypeStruct((N, 1), jnp.float32),
            jax.ShapeDtypeStruct((NS, 1, EPT), jnp.float32),
        ),
    )(z, W, b.reshape(1, 1), edge_attr)
    flat = _sc_scatter(
        u2.reshape(N),
        v2.reshape(N),
        row.reshape(NS, EPT),
        col.reshape(NS, EPT),
        es3.reshape(NS, EPT),
    )
    return (
        flat.reshape(N // 8, N // 128, 8, 128)
        .transpose(0, 2, 1, 3)
        .reshape(N, N)
    )


# speedup vs baseline: 9.4155x; 1.0698x over previous
import functools

import jax
import jax.numpy as jnp
from jax import lax
from jax.experimental import pallas as pl
from jax.experimental.pallas import tpu as pltpu
from jax.experimental.pallas import tpu_sc as plsc

N = 4096
E = 131072
LAT = 256
D_EDGE = 16
NC = 2
NS = 16
L = 16

EPT = E // NS
CH = 128
NJ = EPT // CH
ROWS_PER_BLK = 256
BLK_WORDS = ROWS_PER_BLK * N
BLKS_PER_SC = N // ROWS_PER_BLK // NC
SLICE = BLK_WORDS // NS
ZCH = 8192


def _tc_prep(z_ref, w_ref, b_ref, ea_ref, u_ref, v_ref, es_ref):
    @pl.when(pl.program_id(0) == 0)
    def _():
        zz = z_ref[...]
        w = w_ref[...]
        u_ref[...] = (
            jnp.dot(zz, w[:LAT, :], preferred_element_type=jnp.float32)
            + b_ref[...]
        )
        v_ref[...] = jnp.dot(
            zz, w[LAT:, :], preferred_element_type=jnp.float32
        )

    ones = jnp.ones((1, D_EDGE), jnp.float32)
    es_ref[...] = lax.dot_general(
        ones,
        ea_ref[...],
        dimension_numbers=(((1,), (1,)), ((), ())),
        preferred_element_type=jnp.float32,
    ).reshape(1, 1, EPT)


_mesh = plsc.VectorSubcoreMesh(core_axis_name="c", subcore_axis_name="s")


@functools.partial(
    pl.kernel,
    mesh=_mesh,
    out_type=jax.ShapeDtypeStruct((N * N,), jnp.float32),
    compiler_params=pltpu.CompilerParams(needs_layout_passes=False),
    scratch_types=[
        pltpu.VMEM_SHARED((BLK_WORDS,), jnp.float32),
        pltpu.VMEM((N,), jnp.float32),
        pltpu.VMEM((N,), jnp.float32),
        pltpu.VMEM((EPT,), jnp.int32),
        pltpu.VMEM((EPT,), jnp.int32),
        pltpu.VMEM((EPT,), jnp.int32),
        pltpu.VMEM((EPT,), jnp.float32),
        pltpu.VMEM((EPT,), jnp.float32),
        pltpu.VMEM((ZCH,), jnp.float32),
        pltpu.SemaphoreType.DMA,
        pltpu.SemaphoreType.DMA,
    ],
)
def _sc_scatter(u_hbm, v_hbm, row_hbm, col_hbm, es_hbm, out_hbm,
                shared, u_ts, v_ts, idx_ts, col_ts, k_ts,
                p_ts, es_ts, zero_ts, zsem, wsem):
    c = lax.axis_index("c")
    s = lax.axis_index("s")

    pltpu.sync_copy(u_hbm, u_ts)
    pltpu.sync_copy(v_hbm, v_ts)
    pltpu.sync_copy(row_hbm.at[s], idx_ts)
    pltpu.sync_copy(col_hbm.at[s], col_ts)
    pltpu.sync_copy(es_hbm.at[s], es_ts)

    zeros16 = jnp.zeros((L,), jnp.float32)

    def zbody(i, _):
        zero_ts[pl.ds(i * L, L)] = zeros16
        return _

    lax.fori_loop(0, ZCH // L, zbody, None)

    def prep_body(i, _):
        o = i * L
        rv = idx_ts[pl.ds(o, L)]
        cv = col_ts[pl.ds(o, L)]
        k_ts[pl.ds(o, L)] = rv * N + cv
        uv = plsc.load_gather(u_ts, [rv])
        vv = plsc.load_gather(v_ts, [cv])
        p_ts[pl.ds(o, L)] = uv + vv
        return _

    lax.fori_loop(0, EPT // L, prep_body, None)

    wb_prev = None
    for jj in range(BLKS_PER_SC):
        blk = c * BLKS_PER_SC + jj
        base = blk * BLK_WORDS

        def idx_body(i, _):
            o = i * L
            kv = k_ts[pl.ds(o, L)]
            inblk = (kv >> 20) == blk
            rl = (kv >> 12) & (ROWS_PER_BLK - 1)
            cc = kv & (N - 1)
            local = (
                ((rl >> 3) << 15)
                + ((cc >> 7) << 10)
                + ((rl & 7) << 7)
                + (cc & 127)
            )
            idx_ts[pl.ds(o, L)] = jnp.where(inblk, local, -1)
            return _

        lax.fori_loop(0, EPT // L, idx_body, None)

        if wb_prev is not None:
            wb_prev.wait()
        zdescs = [
            pltpu.async_copy(
                zero_ts, shared.at[pl.ds(s * SLICE + i * ZCH, ZCH)], zsem
            )
            for i in range(SLICE // ZCH)
        ]
        for d in zdescs:
            d.wait()
        plsc.subcore_barrier()

        filtered = plsc.Indices(idx_ts, ignored_value=-1)
        pltpu.sync_copy(p_ts, shared.at[filtered])
        plsc.subcore_barrier()

        pltpu.sync_copy(es_ts, shared.at[filtered], add=True)
        plsc.subcore_barrier()

        wb_prev = pltpu.async_copy(
            shared.at[pl.ds(s * SLICE, SLICE)],
            out_hbm.at[pl.ds(base + s * SLICE, SLICE)],
            wsem,
        )
    wb_prev.wait()


def kernel(z, edge_index, edge_attr, W, b):
    row = edge_index[0].astype(jnp.int32)
    col = edge_index[1].astype(jnp.int32)
    u2, v2, es3 = pl.pallas_call(
        _tc_prep,
        grid=(NS,),
        in_specs=[
            pl.BlockSpec((N, LAT), lambda i: (0, 0)),
            pl.BlockSpec((2 * LAT, 1), lambda i: (0, 0)),
            pl.BlockSpec((1, 1), lambda i: (0, 0)),
            pl.BlockSpec((EPT, D_EDGE), lambda i: (i, 0)),
        ],
        out_specs=[
            pl.BlockSpec((N, 1), lambda i: (0, 0)),
            pl.BlockSpec((N, 1), lambda i: (0, 0)),
            pl.BlockSpec((1, 1, EPT), lambda i: (i, 0, 0)),
        ],
        out_shape=(
            jax.ShapeDtypeStruct((N, 1), jnp.float32),
            jax.ShapeDtypeStruct((N, 1), jnp.float32),
            jax.ShapeDtypeStruct((NS, 1, EPT), jnp.float32),
        ),
    )(z, W, b.reshape(1, 1), edge_attr)
    flat = _sc_scatter(
        u2.reshape(N),
        v2.reshape(N),
        row.reshape(NS, EPT),
        col.reshape(NS, EPT),
        es3.reshape(NS, EPT),
    )
    return (
        flat.reshape(N // 8, N // 128, 8, 128)
        .transpose(0, 2, 1, 3)
        .reshape(N, N)
    )
